# baseline (speedup 1.0000x reference)
import jax
import jax.numpy as jnp
from jax.experimental import pallas as pl
from jax.experimental.pallas import tpu as pltpu

L = 8192
D = 64
SUB = 64
LANE = 128
CBLK = 1024
NBLK = L // CBLK
WIN = 1040
OPAD = L + WIN


def _stage1_body(xr_ref, w_ref, valm_ref, val2_ref, idx_ref):
    xr = xr_ref[...].astype(jnp.bfloat16).astype(jnp.float32)
    w = w_ref[...].astype(jnp.bfloat16).astype(jnp.float32)
    a = jax.nn.sigmoid(jnp.sum(xr * w, axis=2))
    r = jax.lax.broadcasted_iota(jnp.int32, (LANE, LANE), 0)
    c = jax.lax.broadcasted_iota(jnp.int32, (LANE, LANE), 1)
    ut = (r <= c).astype(jnp.float32)
    crow = jax.lax.dot_general(a, ut, (((1,), (0,)), ((), ())),
                               preferred_element_type=jnp.float32,
                               precision=jax.lax.Precision.HIGHEST)
    tot = crow[:, LANE - 1:LANE]
    rs = jax.lax.broadcasted_iota(jnp.int32, (SUB, SUB), 0)
    cs = jax.lax.broadcasted_iota(jnp.int32, (SUB, SUB), 1)
    lt = (cs < rs).astype(jnp.float32)
    offs = jax.lax.dot_general(lt, tot, (((1,), (0,)), ((), ())),
                               preferred_element_type=jnp.float32,
                               precision=jax.lax.Precision.HIGHEST)
    b = crow + offs
    idx = jnp.floor(b).astype(jnp.int32)
    frac = b - idx.astype(jnp.float32)
    prev_idx = jnp.floor(b - a).astype(jnp.int32)
    cross = idx != prev_idx
    valm_ref[...] = jnp.where(cross, frac, a)
    val2_ref[...] = jnp.where(cross, a - frac, jnp.float32(0.0))
    idx_ref[...] = idx


def _stage2_body(x_ref, idx_ref, valm_ref, val2_ref, rs_ref, out_ref):
    k = pl.program_id(0)

    @pl.when(k == 0)
    def _init():
        out_ref[...] = jnp.zeros_like(out_ref)

    base = jnp.maximum(rs_ref[k], 1) - 1
    acc = jnp.zeros((WIN, D), dtype=jnp.float32)
    riota = jax.lax.broadcasted_iota(jnp.int32, (WIN, LANE), 0)
    for s in range(CBLK // LANE):
        t = idx_ref[0, s:s + 1, :] - base
        vm = valm_ref[0, s:s + 1, :]
        v2 = val2_ref[0, s:s + 1, :]
        wmat = (jnp.where(riota == t, vm, jnp.float32(0.0)) +
                jnp.where(riota == t - 1, v2, jnp.float32(0.0)))
        acc = acc + jax.lax.dot_general(
            wmat, x_ref[s * LANE:(s + 1) * LANE, :],
            (((1,), (0,)), ((), ())),
            preferred_element_type=jnp.float32,
                               precision=jax.lax.Precision.HIGHEST)
    cur = out_ref[pl.ds(base, WIN), :]
    out_ref[pl.ds(base, WIN), :] = cur + acc


def kernel(x, w):
    xr = x.reshape(SUB, LANE, D)
    wr = w.reshape(1, 1, D)
    valm, val2, idx = pl.pallas_call(
        _stage1_body,
        out_shape=[
            jax.ShapeDtypeStruct((SUB, LANE), jnp.float32),
            jax.ShapeDtypeStruct((SUB, LANE), jnp.float32),
            jax.ShapeDtypeStruct((SUB, LANE), jnp.int32),
        ],
    )(xr, wr)

    rs = jnp.min(idx.reshape(NBLK, CBLK), axis=1).astype(jnp.int32)

    idx3 = idx.reshape(NBLK, CBLK // LANE, LANE)
    valm3 = valm.reshape(NBLK, CBLK // LANE, LANE)
    val23 = val2.reshape(NBLK, CBLK // LANE, LANE)
    out = pl.pallas_call(
        _stage2_body,
        grid=(NBLK,),
        in_specs=[
            pl.BlockSpec((CBLK, D), lambda k: (k, 0)),
            pl.BlockSpec((1, CBLK // LANE, LANE), lambda k: (k, 0, 0)),
            pl.BlockSpec((1, CBLK // LANE, LANE), lambda k: (k, 0, 0)),
            pl.BlockSpec((1, CBLK // LANE, LANE), lambda k: (k, 0, 0)),
            pl.BlockSpec(memory_space=pltpu.SMEM),
        ],
        out_specs=pl.BlockSpec((OPAD, D), lambda k: (0, 0)),
        out_shape=jax.ShapeDtypeStruct((OPAD, D), jnp.float32),
    )(x, idx3, valm3, val23, rs)
    return out[:L]


# speedup vs baseline: 1.3593x; 1.3593x over previous
import jax
import jax.numpy as jnp
from jax.experimental import pallas as pl
from jax.experimental.pallas import tpu as pltpu

L = 8192
D = 64
SUB = 64
LANE = 128
CBLK = 512
NBLK = L // CBLK
WIN = 528
OPAD = L + WIN


def _stage1_body(xr_ref, w_ref, valm_ref, val2_ref, idx_ref):
    xr = xr_ref[...].astype(jnp.bfloat16).astype(jnp.float32)
    w = w_ref[...].astype(jnp.bfloat16).astype(jnp.float32)
    a = jax.nn.sigmoid(jnp.sum(xr * w, axis=2))
    r = jax.lax.broadcasted_iota(jnp.int32, (LANE, LANE), 0)
    c = jax.lax.broadcasted_iota(jnp.int32, (LANE, LANE), 1)
    ut = (r <= c).astype(jnp.float32)
    crow = jax.lax.dot_general(a, ut, (((1,), (0,)), ((), ())),
                               preferred_element_type=jnp.float32,
                               precision=jax.lax.Precision.HIGHEST)
    tot = crow[:, LANE - 1:LANE]
    rs = jax.lax.broadcasted_iota(jnp.int32, (SUB, SUB), 0)
    cs = jax.lax.broadcasted_iota(jnp.int32, (SUB, SUB), 1)
    lt = (cs < rs).astype(jnp.float32)
    offs = jax.lax.dot_general(lt, tot, (((1,), (0,)), ((), ())),
                               preferred_element_type=jnp.float32,
                               precision=jax.lax.Precision.HIGHEST)
    b = crow + offs
    idx = jnp.floor(b).astype(jnp.int32)
    frac = b - idx.astype(jnp.float32)
    prev_idx = jnp.floor(b - a).astype(jnp.int32)
    cross = idx != prev_idx
    valm_ref[...] = jnp.where(cross, frac, a)
    val2_ref[...] = jnp.where(cross, a - frac, jnp.float32(0.0))
    idx_ref[...] = idx


def _stage2_body(x_ref, idx_ref, valm_ref, val2_ref, rs_ref, out_ref):
    k = pl.program_id(0)

    @pl.when(k == 0)
    def _init():
        out_ref[...] = jnp.zeros_like(out_ref)

    base = jnp.maximum(rs_ref[k], 1) - 1
    acc = jnp.zeros((WIN, D), dtype=jnp.float32)
    riota = jax.lax.broadcasted_iota(jnp.int32, (WIN, LANE), 0)
    for s in range(CBLK // LANE):
        t = idx_ref[0, s:s + 1, :] - base
        vm = valm_ref[0, s:s + 1, :]
        v2 = val2_ref[0, s:s + 1, :]
        wmat = (jnp.where(riota == t, vm, jnp.float32(0.0)) +
                jnp.where(riota == t - 1, v2, jnp.float32(0.0)))
        acc = acc + jax.lax.dot_general(
            wmat, x_ref[s * LANE:(s + 1) * LANE, :],
            (((1,), (0,)), ((), ())),
            preferred_element_type=jnp.float32,
                               precision=jax.lax.Precision.HIGHEST)
    cur = out_ref[pl.ds(base, WIN), :]
    out_ref[pl.ds(base, WIN), :] = cur + acc


def kernel(x, w):
    xr = x.reshape(SUB, LANE, D)
    wr = w.reshape(1, 1, D)
    valm, val2, idx = pl.pallas_call(
        _stage1_body,
        out_shape=[
            jax.ShapeDtypeStruct((SUB, LANE), jnp.float32),
            jax.ShapeDtypeStruct((SUB, LANE), jnp.float32),
            jax.ShapeDtypeStruct((SUB, LANE), jnp.int32),
        ],
    )(xr, wr)

    rs = jnp.min(idx.reshape(NBLK, CBLK), axis=1).astype(jnp.int32)

    idx3 = idx.reshape(NBLK, CBLK // LANE, LANE)
    valm3 = valm.reshape(NBLK, CBLK // LANE, LANE)
    val23 = val2.reshape(NBLK, CBLK // LANE, LANE)
    out = pl.pallas_call(
        _stage2_body,
        grid=(NBLK,),
        in_specs=[
            pl.BlockSpec((CBLK, D), lambda k: (k, 0)),
            pl.BlockSpec((1, CBLK // LANE, LANE), lambda k: (k, 0, 0)),
            pl.BlockSpec((1, CBLK // LANE, LANE), lambda k: (k, 0, 0)),
            pl.BlockSpec((1, CBLK // LANE, LANE), lambda k: (k, 0, 0)),
            pl.BlockSpec(memory_space=pltpu.SMEM),
        ],
        out_specs=pl.BlockSpec((OPAD, D), lambda k: (0, 0)),
        out_shape=jax.ShapeDtypeStruct((OPAD, D), jnp.float32),
    )(x, idx3, valm3, val23, rs)
    return out[:L]
